# Initial kernel scaffold; baseline (speedup 1.0000x reference)
import jax
import jax.numpy as jnp
from jax.experimental import pallas as pl


def kernel(pooled, feature_map, ln1_w, ln1_b, gate_scale, gate_bias, residual_scale, ln2_w, ln2_b):
    raise NotImplementedError("write your pallas kernel here")


# speedup vs baseline: 23.6177x; 23.6177x over previous
import functools

import jax
import jax.numpy as jnp
from jax.experimental import pallas as pl

_LN_EPS = 1e-5
_TOPK_RATIO = 0.25


def _to_ukey(x):
    u = jax.lax.bitcast_convert_type(x, jnp.uint32)
    topbit = jnp.uint32(0x80000000)
    return jnp.where(u >= topbit, ~u, u | topbit)


def _from_ukey(p):
    topbit = jnp.uint32(0x80000000)
    u = jnp.where(p >= topbit, p ^ topbit, ~p)
    return jax.lax.bitcast_convert_type(u, jnp.float32)


def _salient_body(x_ref, out_ref, *, k):
    x = x_ref[0]
    ukey = _to_ukey(x)
    kk = jnp.int32(k)

    def step(i, p):
        b = jnp.uint32(31) - jnp.uint32(i)
        cand = p | (jnp.uint32(1) << b)
        cnt = jnp.sum((ukey >= cand).astype(jnp.int32), axis=-1,
                      keepdims=True)
        return jnp.where(cnt >= kk, cand, p)

    p0 = jnp.zeros((x.shape[0], 1), dtype=jnp.uint32)
    p = jax.lax.fori_loop(0, 32, step, p0)

    gt = ukey > p
    cnt_gt = jnp.sum(gt.astype(jnp.int32), axis=-1)
    s_gt = jnp.sum(jnp.where(gt, x, 0.0), axis=-1)
    vk = _from_ukey(p[:, 0])
    total = s_gt + (kk - cnt_gt).astype(jnp.float32) * vk
    out_ref[0, 0] = total * (1.0 / k)


def _fusion_body(pooled_ref, salient_ref, ln1_w_ref, ln1_b_ref,
                 gs_ref, gb_ref, rs_ref, ln2_w_ref, ln2_b_ref, out_ref):
    def ln(v, w, b):
        mu = jnp.mean(v, axis=-1, keepdims=True)
        var = jnp.mean((v - mu) ** 2, axis=-1, keepdims=True)
        return (v - mu) * jax.lax.rsqrt(var + _LN_EPS) * w + b

    pooled = pooled_ref[...]
    salient = ln(salient_ref[...], ln1_w_ref[...], ln1_b_ref[...])
    delta = salient - pooled
    gate = jax.nn.sigmoid(gs_ref[...] * salient + gb_ref[...])
    fused = pooled + rs_ref[...] * gate * delta
    out_ref[...] = ln(fused, ln2_w_ref[...], ln2_b_ref[...])


def kernel(pooled, feature_map, ln1_w, ln1_b, gate_scale, gate_bias,
           residual_scale, ln2_w, ln2_b):
    B, C, H, W = feature_map.shape
    N = H * W
    k = max(1, min(N, int(round(N * _TOPK_RATIO))))
    fm = feature_map.astype(jnp.float32).reshape(B, C, N)

    salient = pl.pallas_call(
        functools.partial(_salient_body, k=k),
        grid=(B,),
        in_specs=[pl.BlockSpec((1, C, N), lambda b: (b, 0, 0))],
        out_specs=pl.BlockSpec((1, 1, C), lambda b: (b, 0, 0)),
        out_shape=jax.ShapeDtypeStruct((B, 1, C), jnp.float32),
    )(fm).reshape(B, C)

    params = [p.reshape(1, C) for p in
              (ln1_w, ln1_b, gate_scale, gate_bias, residual_scale,
               ln2_w, ln2_b)]
    out = pl.pallas_call(
        _fusion_body,
        out_shape=jax.ShapeDtypeStruct((B, C), jnp.float32),
    )(pooled.astype(jnp.float32), salient, *params)
    return out.astype(pooled.dtype)
